# baseline (speedup 1.0000x reference)
import jax
import jax.numpy as jnp
from jax import lax
from jax.experimental import pallas as pl
from jax.experimental.pallas import tpu as pltpu
from jax.experimental.pallas import tpu_sc as plsc

KERNEL_SIZE = 1024
MAX_DEPTH = 17
N_POINTS = 524288
D_FEAT = 64
SCALE = float(2 ** MAX_DEPTH)

_NC = 2
_NS = 16
_NW = _NC * _NS
_CHUNK = N_POINTS // _NW
_LANES = 16


def _sc_gather_body(pxy_hbm, w_hbm, b_hbm, wout_hbm, bout_hbm,
                    pxy_v, wo_v, bo_v, wt_v, bt_v):
    wid = lax.axis_index("s") * _NC + lax.axis_index("c")
    base = wid * _CHUNK
    pltpu.sync_copy(w_hbm, wt_v)
    pltpu.sync_copy(b_hbm, bt_v)
    pltpu.sync_copy(pxy_hbm.at[pl.ds(2 * base, 2 * _CHUNK)], pxy_v)

    @plsc.parallel_loop(0, 2 * _CHUNK, step=256, unroll=2)
    def body(o2):
        so = lax.shift_right_logical(o2, 1)
        for j in range(128 // _LANES):
            px = pxy_v[pl.ds(o2 + j * _LANES, _LANES)]
            py = pxy_v[pl.ds(o2 + 128 + j * _LANES, _LANES)]
            fx = jnp.minimum(jnp.maximum(px * SCALE, 0.0), SCALE - 1.0)
            fy = jnp.minimum(jnp.maximum(py * SCALE, 0.0), SCALE - 1.0)
            ix = fx.astype(jnp.int32)
            iy = fy.astype(jnp.int32)
            bucket = ((ix & (KERNEL_SIZE - 1)) * 31 + iy) & (KERNEL_SIZE - 1)
            wo_v[pl.ds(so + j * _LANES, _LANES)] = plsc.load_gather(
                wt_v, [bucket])
            bo_v[pl.ds(so + j * _LANES, _LANES)] = plsc.load_gather(
                bt_v, [bucket])

    pltpu.sync_copy(wo_v, wout_hbm.at[pl.ds(base, _CHUNK)])
    pltpu.sync_copy(bo_v, bout_hbm.at[pl.ds(base, _CHUNK)])


def _sc_gather(pxy, weight, bias):
    mesh = plsc.VectorSubcoreMesh(core_axis_name="c", subcore_axis_name="s")
    fn = pl.kernel(
        _sc_gather_body,
        mesh=mesh,
        compiler_params=pltpu.CompilerParams(needs_layout_passes=False),
        out_type=[
            jax.ShapeDtypeStruct((N_POINTS,), jnp.float32),
            jax.ShapeDtypeStruct((N_POINTS,), jnp.float32),
        ],
        scratch_types=[
            pltpu.VMEM((2 * _CHUNK,), jnp.float32),
            pltpu.VMEM((_CHUNK,), jnp.float32),
            pltpu.VMEM((_CHUNK,), jnp.float32),
            pltpu.VMEM((KERNEL_SIZE,), jnp.float32),
            pltpu.VMEM((KERNEL_SIZE,), jnp.float32),
        ],
    )
    return fn(pxy, weight, bias)


def _affine_body(x_ref, w_ref, b_ref, o_ref):
    w = w_ref[...][None, :]
    b = b_ref[...][None, :]
    o_ref[...] = w * x_ref[...] + b


def _affine_t(xt, w, b):
    bn = 32768
    grid = (N_POINTS // bn,)
    return pl.pallas_call(
        _affine_body,
        grid=grid,
        in_specs=[
            pl.BlockSpec((D_FEAT, bn), lambda i: (0, i)),
            pl.BlockSpec((bn,), lambda i: (i,)),
            pl.BlockSpec((bn,), lambda i: (i,)),
        ],
        out_specs=pl.BlockSpec((D_FEAT, bn), lambda i: (0, i)),
        out_shape=jax.ShapeDtypeStruct((D_FEAT, N_POINTS), jnp.float32),
    )(xt, w, b)


def kernel(input, x, weight, bias):
    pxy = input.reshape(N_POINTS // 128, 128, 2).transpose(0, 2, 1).reshape(
        2 * N_POINTS)
    w_pts, b_pts = _sc_gather(pxy, weight, bias)
    out_t = _affine_t(x.T, w_pts, b_pts)
    return out_t.T


# speedup vs baseline: 1.0143x; 1.0143x over previous
import jax
import jax.numpy as jnp
from jax import lax
from jax.experimental import pallas as pl
from jax.experimental.pallas import tpu as pltpu
from jax.experimental.pallas import tpu_sc as plsc

KERNEL_SIZE = 1024
MAX_DEPTH = 17
N_POINTS = 524288
D_FEAT = 64
SCALE = float(2 ** MAX_DEPTH)

_NC = 2
_NS = 16
_NW = _NC * _NS
_CHUNK = N_POINTS // _NW
_LANES = 16


def _sc_gather_body(pxy_hbm, w_hbm, b_hbm, wout_hbm, bout_hbm,
                    pxy_v, wo_v, bo_v, wt_v, bt_v):
    wid = lax.axis_index("s") * _NC + lax.axis_index("c")
    base = wid * _CHUNK
    pltpu.sync_copy(w_hbm, wt_v)
    pltpu.sync_copy(b_hbm, bt_v)
    pltpu.sync_copy(pxy_hbm.at[pl.ds(2 * base, 2 * _CHUNK)], pxy_v)

    @plsc.parallel_loop(0, _CHUNK, step=_LANES, unroll=16)
    def body(o):
        goff = (o >> 7) * 256 + (o & 127)
        px = pxy_v[pl.ds(goff, _LANES)]
        py = pxy_v[pl.ds(goff + 128, _LANES)]
        fx = jnp.minimum(jnp.maximum(px * SCALE, 0.0), SCALE - 1.0)
        fy = jnp.minimum(jnp.maximum(py * SCALE, 0.0), SCALE - 1.0)
        ix = fx.astype(jnp.int32)
        iy = fy.astype(jnp.int32)
        bucket = ((ix & (KERNEL_SIZE - 1)) * 31 + iy) & (KERNEL_SIZE - 1)
        wo_v[pl.ds(o, _LANES)] = plsc.load_gather(wt_v, [bucket])
        bo_v[pl.ds(o, _LANES)] = plsc.load_gather(bt_v, [bucket])

    pltpu.sync_copy(wo_v, wout_hbm.at[pl.ds(base, _CHUNK)])
    pltpu.sync_copy(bo_v, bout_hbm.at[pl.ds(base, _CHUNK)])


def _sc_gather(pxy, weight, bias):
    mesh = plsc.VectorSubcoreMesh(core_axis_name="c", subcore_axis_name="s")
    fn = pl.kernel(
        _sc_gather_body,
        mesh=mesh,
        compiler_params=pltpu.CompilerParams(needs_layout_passes=False),
        out_type=[
            jax.ShapeDtypeStruct((N_POINTS,), jnp.float32),
            jax.ShapeDtypeStruct((N_POINTS,), jnp.float32),
        ],
        scratch_types=[
            pltpu.VMEM((2 * _CHUNK,), jnp.float32),
            pltpu.VMEM((_CHUNK,), jnp.float32),
            pltpu.VMEM((_CHUNK,), jnp.float32),
            pltpu.VMEM((KERNEL_SIZE,), jnp.float32),
            pltpu.VMEM((KERNEL_SIZE,), jnp.float32),
        ],
    )
    return fn(pxy, weight, bias)


def _affine_body(x_ref, w_ref, b_ref, o_ref):
    w = w_ref[...][None, :]
    b = b_ref[...][None, :]
    o_ref[...] = w * x_ref[...] + b


def _affine_t(xt, w, b):
    bn = 32768
    grid = (N_POINTS // bn,)
    return pl.pallas_call(
        _affine_body,
        grid=grid,
        in_specs=[
            pl.BlockSpec((D_FEAT, bn), lambda i: (0, i)),
            pl.BlockSpec((bn,), lambda i: (i,)),
            pl.BlockSpec((bn,), lambda i: (i,)),
        ],
        out_specs=pl.BlockSpec((D_FEAT, bn), lambda i: (0, i)),
        out_shape=jax.ShapeDtypeStruct((D_FEAT, N_POINTS), jnp.float32),
    )(xt, w, b)


def kernel(input, x, weight, bias):
    pxy = input.reshape(N_POINTS // 128, 128, 2).transpose(0, 2, 1).reshape(
        2 * N_POINTS)
    w_pts, b_pts = _sc_gather(pxy, weight, bias)
    out_t = _affine_t(x.T, w_pts, b_pts)
    return out_t.T
